# baseline (speedup 1.0000x reference)
import jax
import jax.numpy as jnp
from jax.experimental import pallas as pl
from jax.experimental.pallas import tpu as pltpu

_BI = 400
_BK = 2048


def _make_body(N, NPAD, K):
    last_valid = N - (K - 1) * _BK

    def body(adj_ref, feat16_ref, featblk_ref, w1_ref, w2_ref, bias_ref,
             pre_ref, out_ref, acc_ref, cache_ref, g_ref, pre1_ref):
        p = pl.program_id(0)
        i = pl.program_id(1)
        k = pl.program_id(2)
        dn = (((1,), (1,)), ((), ()))

        def masked_a16():
            col = jax.lax.broadcasted_iota(jnp.int32, (_BI, _BK), 1)
            return jnp.where(col < last_valid, adj_ref[...], 0.0).astype(
                jnp.bfloat16)

        @pl.when(p == 0)
        def _pass1():
            @pl.when((i == 0) & (k == 0))
            def _():
                g_ref[pl.ds(N, NPAD - N), :] = jnp.zeros(
                    (NPAD - N, g_ref.shape[1]), jnp.bfloat16)

            @pl.when(k == 0)
            def _():
                acc_ref[...] = jnp.zeros_like(acc_ref)
                cache_ref[pl.ds(i * _BI, _BI), :] = adj_ref[...].astype(
                    jnp.bfloat16)

            @pl.when(k != K - 1)
            def _():
                acc_ref[...] += jnp.dot(
                    adj_ref[...].astype(jnp.bfloat16),
                    feat16_ref[pl.ds(k * _BK, _BK), :],
                    preferred_element_type=jnp.float32)

            @pl.when(k == K - 1)
            def _():
                acc_ref[...] += jnp.dot(
                    masked_a16(),
                    feat16_ref[pl.ds((K - 1) * _BK, _BK), :],
                    preferred_element_type=jnp.float32)
                ax = acc_ref[...]
                g_ref[pl.ds(i * _BI, _BI), :] = (
                    ax * featblk_ref[...]).astype(jnp.bfloat16)
                pre1_ref[pl.ds(i * _BI, _BI), :] = jax.lax.dot_general(
                    ax, w1_ref[...], dn,
                    precision=jax.lax.Precision.HIGHEST,
                    preferred_element_type=jnp.float32).astype(jnp.bfloat16)

        @pl.when(p == 1)
        def _pass2():
            @pl.when(k == 0)
            def _():
                acc_ref[...] = jnp.dot(
                    cache_ref[pl.ds(i * _BI, _BI), :],
                    g_ref[pl.ds(0, _BK), :],
                    preferred_element_type=jnp.float32)

            @pl.when((k > 0) & (k != K - 1))
            def _():
                acc_ref[...] += jnp.dot(
                    adj_ref[...].astype(jnp.bfloat16),
                    g_ref[pl.ds(k * _BK, _BK), :],
                    preferred_element_type=jnp.float32)

            @pl.when(k == K - 1)
            def _():
                acc_ref[...] += jnp.dot(
                    masked_a16(),
                    g_ref[pl.ds((K - 1) * _BK, _BK), :],
                    preferred_element_type=jnp.float32)
                pre = (
                    pre1_ref[pl.ds(i * _BI, _BI), :].astype(jnp.float32)
                    + jax.lax.dot_general(
                        acc_ref[...], w2_ref[...], dn,
                        precision=jax.lax.Precision.HIGHEST,
                        preferred_element_type=jnp.float32)
                    + bias_ref[...]
                )
                pre_ref[...] = pre
                out_ref[...] = jnp.where(pre > 0, pre, jnp.exp(pre) - 1.0)

    return body


def kernel(features, adj, W1, b1, W2, b2):
    N, H = features.shape
    R = N // _BI
    K = -(-N // _BK)
    NPAD = K * _BK
    feat16 = jnp.pad(features.astype(jnp.bfloat16), ((0, NPAD - N), (0, 0)))
    bias = (b1 + b2).reshape(1, H)

    pre, out = pl.pallas_call(
        _make_body(N, NPAD, K),
        grid=(2, R, K),
        in_specs=[
            pl.BlockSpec((_BI, _BK), lambda p, i, k: (i, jnp.maximum(k, p))),
            pl.BlockSpec((NPAD, H), lambda p, i, k: (0, 0)),
            pl.BlockSpec((_BI, H), lambda p, i, k: (i * (1 - p), 0)),
            pl.BlockSpec((H, H), lambda p, i, k: (0, 0)),
            pl.BlockSpec((H, H), lambda p, i, k: (0, 0)),
            pl.BlockSpec((1, H), lambda p, i, k: (0, 0)),
        ],
        out_specs=[
            pl.BlockSpec((_BI, H), lambda p, i, k: (i * p, 0)),
            pl.BlockSpec((_BI, H), lambda p, i, k: (i * p, 0)),
        ],
        out_shape=[
            jax.ShapeDtypeStruct((N, H), jnp.float32),
            jax.ShapeDtypeStruct((N, H), jnp.float32),
        ],
        scratch_shapes=[
            pltpu.VMEM((_BI, H), jnp.float32),
            pltpu.VMEM((N, _BK), jnp.bfloat16),
            pltpu.VMEM((NPAD, H), jnp.bfloat16),
            pltpu.VMEM((N, H), jnp.bfloat16),
        ],
    )(adj, feat16, features, W1, W2, bias)
    return (pre, out)


# speedup vs baseline: 1.2917x; 1.2917x over previous
import jax
import jax.numpy as jnp
from jax.experimental import pallas as pl
from jax.experimental.pallas import tpu as pltpu

_BI = 200
_RC = 7


def _body(adj_ref, feat16_ref, w1_ref, w2_ref, bias_ref,
          pre_ref, out_ref, cache_ref, g_ref, pre1_ref):
    p = pl.program_id(0)
    i = pl.program_id(1)
    dn = (((1,), (1,)), ((), ()))

    @pl.when(p == 0)
    def _pass1():
        a16 = adj_ref[...].astype(jnp.bfloat16)

        @pl.when(i < _RC)
        def _():
            cache_ref[pl.ds(i * _BI, _BI), :] = a16

        ax = jnp.dot(a16, feat16_ref[...], preferred_element_type=jnp.float32)
        g_ref[pl.ds(i * _BI, _BI), :] = (
            ax * feat16_ref[pl.ds(i * _BI, _BI), :].astype(jnp.float32)
        ).astype(jnp.bfloat16)
        pre1_ref[pl.ds(i * _BI, _BI), :] = jax.lax.dot_general(
            ax, w1_ref[...], dn,
            precision=jax.lax.Precision.HIGHEST,
            preferred_element_type=jnp.float32).astype(jnp.bfloat16)

    @pl.when(p == 1)
    def _pass2():
        def finish(axx):
            pre = (
                pre1_ref[pl.ds(i * _BI, _BI), :].astype(jnp.float32)
                + jax.lax.dot_general(
                    axx, w2_ref[...], dn,
                    precision=jax.lax.Precision.HIGHEST,
                    preferred_element_type=jnp.float32)
                + bias_ref[...]
            )
            pre_ref[...] = pre
            out_ref[...] = jnp.where(pre > 0, pre, jnp.exp(pre) - 1.0)

        @pl.when(i < _RC)
        def _():
            finish(jnp.dot(cache_ref[pl.ds(i * _BI, _BI), :], g_ref[...],
                           preferred_element_type=jnp.float32))

        @pl.when(i >= _RC)
        def _():
            finish(jnp.dot(adj_ref[...].astype(jnp.bfloat16), g_ref[...],
                           preferred_element_type=jnp.float32))


def kernel(features, adj, W1, b1, W2, b2):
    N, H = features.shape
    R = N // _BI
    feat16 = features.astype(jnp.bfloat16)
    bias = (b1 + b2).reshape(1, H)

    pre, out = pl.pallas_call(
        _body,
        grid=(2, R),
        in_specs=[
            pl.BlockSpec((_BI, N), lambda p, i: (jnp.maximum(i, p * _RC), 0)),
            pl.BlockSpec((N, H), lambda p, i: (0, 0)),
            pl.BlockSpec((H, H), lambda p, i: (0, 0)),
            pl.BlockSpec((H, H), lambda p, i: (0, 0)),
            pl.BlockSpec((1, H), lambda p, i: (0, 0)),
        ],
        out_specs=[
            pl.BlockSpec((_BI, H), lambda p, i: (i * p, 0)),
            pl.BlockSpec((_BI, H), lambda p, i: (i * p, 0)),
        ],
        out_shape=[
            jax.ShapeDtypeStruct((N, H), jnp.float32),
            jax.ShapeDtypeStruct((N, H), jnp.float32),
        ],
        scratch_shapes=[
            pltpu.VMEM((_RC * _BI, N), jnp.bfloat16),
            pltpu.VMEM((N, H), jnp.bfloat16),
            pltpu.VMEM((N, H), jnp.bfloat16),
        ],
    )(adj, feat16, W1, W2, bias)
    return (pre, out)


# speedup vs baseline: 1.3136x; 1.0170x over previous
import jax
import jax.numpy as jnp
from jax.experimental import pallas as pl

_BI = 400


def _pass1_body(adj_ref, featfull_ref, featblk_ref, ax_ref, g_ref):
    ax = jnp.dot(adj_ref[...], featfull_ref[...],
                 preferred_element_type=jnp.float32)
    ax_ref[...] = ax
    g_ref[...] = ax * featblk_ref[...]


def _pass2_body(adj_ref, gfull_ref, ax_ref, w1_ref, w2_ref, bias_ref,
                pre_ref, out_ref):
    axx = jnp.dot(adj_ref[...], gfull_ref[...],
                  preferred_element_type=jnp.float32)
    dn = (((1,), (1,)), ((), ()))
    pre = (
        jax.lax.dot_general(ax_ref[...], w1_ref[...], dn,
                            precision=jax.lax.Precision.HIGHEST,
                            preferred_element_type=jnp.float32)
        + jax.lax.dot_general(axx, w2_ref[...], dn,
                              precision=jax.lax.Precision.HIGHEST,
                              preferred_element_type=jnp.float32)
        + bias_ref[...]
    )
    pre_ref[...] = pre
    out_ref[...] = jnp.where(pre > 0, pre, jnp.exp(pre) - 1.0)


def kernel(features, adj, W1, b1, W2, b2):
    N, H = features.shape
    R = N // _BI
    bias = (b1 + b2).reshape(1, H)

    ax, g = pl.pallas_call(
        _pass1_body,
        grid=(R,),
        in_specs=[
            pl.BlockSpec((_BI, N), lambda i: (i, 0)),
            pl.BlockSpec((N, H), lambda i: (0, 0)),
            pl.BlockSpec((_BI, H), lambda i: (i, 0)),
        ],
        out_specs=[
            pl.BlockSpec((_BI, H), lambda i: (i, 0)),
            pl.BlockSpec((_BI, H), lambda i: (i, 0)),
        ],
        out_shape=[
            jax.ShapeDtypeStruct((N, H), jnp.float32),
            jax.ShapeDtypeStruct((N, H), jnp.float32),
        ],
    )(adj, features, features)

    pre, out = pl.pallas_call(
        _pass2_body,
        grid=(R,),
        in_specs=[
            pl.BlockSpec((_BI, N), lambda i: (i, 0)),
            pl.BlockSpec((N, H), lambda i: (0, 0)),
            pl.BlockSpec((_BI, H), lambda i: (i, 0)),
            pl.BlockSpec((H, H), lambda i: (0, 0)),
            pl.BlockSpec((H, H), lambda i: (0, 0)),
            pl.BlockSpec((1, H), lambda i: (0, 0)),
        ],
        out_specs=[
            pl.BlockSpec((_BI, H), lambda i: (i, 0)),
            pl.BlockSpec((_BI, H), lambda i: (i, 0)),
        ],
        out_shape=[
            jax.ShapeDtypeStruct((N, H), jnp.float32),
            jax.ShapeDtypeStruct((N, H), jnp.float32),
        ],
    )(adj, g, ax, W1, W2, bias)
    return (pre, out)
